# Initial kernel scaffold; baseline (speedup 1.0000x reference)
import jax
import jax.numpy as jnp
from jax.experimental import pallas as pl


def kernel(edge_index, features, preference, W1, b1, W2, b2):
    raise NotImplementedError("write your pallas kernel here")


# speedup vs baseline: 5.0298x; 5.0298x over previous
import functools

import jax
import jax.numpy as jnp
from jax import lax
from jax.experimental import pallas as pl
from jax.experimental.pallas import tpu as pltpu
from jax.experimental.pallas import tpu_sc as plsc

NUM_USER = 10000
NUM_ITEM = 40000
N_NODES = NUM_USER + NUM_ITEM
DIM = 64
DIM_Q = 16
FEAT = 128
HID = 256

N_EDGES = 800000
CH = 128
NC, NS = 2, 16
E_PAD = 802816
PREP_CHUNKS = E_PAD // (CH * NC * NS)
LAYER_CHUNKS = E_PAD // (CH * NS)

N_PAD = 50176
ROWS_T = N_PAD // NS

_MESH = plsc.VectorSubcoreMesh(core_axis_name="c", subcore_axis_name="s",
                               num_cores=NC, num_subcores=NS)
_SC_PARAMS = pltpu.CompilerParams(use_tc_tiling_on_sc=False)


@functools.partial(
    pl.kernel,
    out_type=[
        jax.ShapeDtypeStruct((E_PAD,), jnp.int32),
        jax.ShapeDtypeStruct((N_PAD,), jnp.float32),
        jax.ShapeDtypeStruct((N_PAD,), jnp.float32),
    ],
    mesh=_MESH,
    compiler_params=_SC_PARAMS,
    scratch_types=[
        pltpu.VMEM((CH,), jnp.int32),
        pltpu.VMEM((CH,), jnp.int32),
        pltpu.VMEM((CH,), jnp.int32),
        pltpu.VMEM((CH,), jnp.float32),
        pltpu.VMEM((ROWS_T,), jnp.float32),
        pltpu.VMEM_SHARED((N_PAD,), jnp.float32),
    ],
)
def _sc_prep(row_hbm, col_hbm, zrow_hbm, rowp_hbm, degp0_hbm, degp1_hbm,
             rv, cv, rpv, ones_v, dbuf, acc1):
    cid = lax.axis_index("c")
    t = lax.axis_index("s")
    wid = cid * NS + t
    dummy = N_NODES + wid

    pltpu.sync_copy(zrow_hbm, dbuf)
    pltpu.sync_copy(dbuf, acc1.at[pl.ds(t * ROWS_T, ROWS_T)])
    plsc.subcore_barrier()

    def chunk(j, _):
        base = (wid * PREP_CHUNKS + j) * CH
        pltpu.sync_copy(row_hbm.at[pl.ds(base, CH)], rv)
        pltpu.sync_copy(col_hbm.at[pl.ds(base, CH)], cv)
        for k in range(CH // 16):
            r = rv[pl.ds(k * 16, 16)]
            c = cv[pl.ds(k * 16, 16)]
            m = r != c
            rpv[pl.ds(k * 16, 16)] = jnp.where(m, r, dummy)
            ones_v[pl.ds(k * 16, 16)] = jnp.where(m, 1.0, 0.0)
        pltpu.sync_copy(rpv, rowp_hbm.at[pl.ds(base, CH)])
        pltpu.sync_copy(ones_v, acc1.at[rv], add=True)
        return ()

    lax.fori_loop(0, PREP_CHUNKS, chunk, ())
    plsc.subcore_barrier()

    pltpu.sync_copy(acc1.at[pl.ds(t * ROWS_T, ROWS_T)], dbuf)
    @pl.when(cid == 0)
    def _():
        pltpu.sync_copy(dbuf, degp0_hbm.at[pl.ds(t * ROWS_T, ROWS_T)])
    @pl.when(cid == 1)
    def _():
        pltpu.sync_copy(dbuf, degp1_hbm.at[pl.ds(t * ROWS_T, ROWS_T)])


@functools.partial(
    pl.kernel,
    out_type=[jax.ShapeDtypeStruct((N_PAD, DIM_Q), jnp.float32)
              for _ in range(4)],
    mesh=_MESH,
    compiler_params=_SC_PARAMS,
    scratch_types=[
        pltpu.VMEM((CH,), jnp.int32),
        pltpu.VMEM((CH,), jnp.int32),
        pltpu.VMEM((CH, DIM_Q), jnp.float32),
        pltpu.VMEM((ROWS_T, DIM_Q), jnp.float32),
        pltpu.VMEM_SHARED((N_PAD, DIM_Q), jnp.float32),
        pltpu.SemaphoreType.DMA,
    ],
)
def _sc_layer(y0_hbm, y1_hbm, y2_hbm, y3_hbm, rowp_hbm, col_hbm, zblk_hbm,
              s0_hbm, s1_hbm, s2_hbm, s3_hbm,
              idx_r, idx_c, gbuf, wbuf, acc, sem):
    cid = lax.axis_index("c")
    t = lax.axis_index("s")
    rows_t = pl.ds(t * ROWS_T, ROWS_T)

    for p in range(2):
        pltpu.sync_copy(zblk_hbm, wbuf)
        pltpu.sync_copy(wbuf, acc.at[rows_t])
        plsc.subcore_barrier()

        def chunk(j, _):
            base = (t * LAYER_CHUNKS + j) * CH
            pltpu.sync_copy(rowp_hbm.at[pl.ds(base, CH)], idx_r)
            pltpu.sync_copy(col_hbm.at[pl.ds(base, CH)], idx_c)
            @pl.when(cid == 0)
            def _():
                src = y0_hbm if p == 0 else y1_hbm
                pltpu.async_copy(src.at[idx_r], gbuf, sem).wait()
            @pl.when(cid == 1)
            def _():
                src = y2_hbm if p == 0 else y3_hbm
                pltpu.async_copy(src.at[idx_r], gbuf, sem).wait()
            pltpu.sync_copy(gbuf, acc.at[idx_c], add=True)
            return ()

        lax.fori_loop(0, LAYER_CHUNKS, chunk, ())
        plsc.subcore_barrier()

        pltpu.sync_copy(acc.at[rows_t], wbuf)
        @pl.when(cid == 0)
        def _():
            dst = s0_hbm if p == 0 else s1_hbm
            pltpu.sync_copy(wbuf, dst.at[rows_t])
        @pl.when(cid == 1)
        def _():
            dst = s2_hbm if p == 0 else s3_hbm
            pltpu.sync_copy(wbuf, dst.at[rows_t])
        plsc.subcore_barrier()


def _degsum_body(p0_ref, p1_ref, dinv_ref, dinv2_ref):
    deg = p0_ref[...] + p1_ref[...]
    dinv_ref[...] = lax.rsqrt(deg)
    dinv2_ref[...] = 1.0 / deg


def _tc_degsum(degp0, degp1):
    return pl.pallas_call(
        _degsum_body,
        out_shape=[
            jax.ShapeDtypeStruct((N_PAD,), jnp.float32),
            jax.ShapeDtypeStruct((N_PAD,), jnp.float32),
        ],
    )(degp0, degp1)


_BM = 2000


def _quarters(y):
    return [y[:, q * DIM_Q:(q + 1) * DIM_Q] for q in range(4)]


def _mlp_body(feat_ref, w1_ref, b1_ref, w2_ref, b2_ref, dinv_ref,
              x_ref, y0_ref, y1_ref, y2_ref, y3_ref):
    t1 = lax.dot_general(feat_ref[...], w1_ref[...],
                         (((1,), (1,)), ((), ())),
                         preferred_element_type=jnp.float32)
    t1 = t1 + b1_ref[...]
    t1 = jnp.where(t1 >= 0, t1, 0.01 * t1)
    x = lax.dot_general(t1, w2_ref[...], (((1,), (1,)), ((), ())),
                        preferred_element_type=jnp.float32)
    x = x + b2_ref[...]
    nrm = jnp.sqrt(jnp.sum(x * x, axis=1, keepdims=True))
    x = x / jnp.maximum(nrm, 1e-12)
    x_ref[...] = x
    y = x * dinv_ref[...]
    for q, ref in enumerate((y0_ref, y1_ref, y2_ref, y3_ref)):
        ref[...] = y[:, q * DIM_Q:(q + 1) * DIM_Q]


def _tc_items(features, W1, b1, W2, b2, dinv_items):
    g = NUM_ITEM // _BM
    qspec = pl.BlockSpec((_BM, DIM_Q), lambda i: (i, 0))
    return pl.pallas_call(
        _mlp_body,
        grid=(g,),
        in_specs=[
            pl.BlockSpec((_BM, FEAT), lambda i: (i, 0)),
            pl.BlockSpec((HID, FEAT), lambda i: (0, 0)),
            pl.BlockSpec((1, HID), lambda i: (0, 0)),
            pl.BlockSpec((DIM, HID), lambda i: (0, 0)),
            pl.BlockSpec((1, DIM), lambda i: (0, 0)),
            pl.BlockSpec((_BM, 1), lambda i: (i, 0)),
        ],
        out_specs=[pl.BlockSpec((_BM, DIM), lambda i: (i, 0))] + [qspec] * 4,
        out_shape=[jax.ShapeDtypeStruct((NUM_ITEM, DIM), jnp.float32)]
        + [jax.ShapeDtypeStruct((NUM_ITEM, DIM_Q), jnp.float32)] * 4,
    )(features, W1, b1.reshape(1, HID), W2, b2.reshape(1, DIM), dinv_items)


def _pref_body(p_ref, dinv_ref, x_ref, y0_ref, y1_ref, y2_ref, y3_ref):
    x = p_ref[...]
    nrm = jnp.sqrt(jnp.sum(x * x, axis=1, keepdims=True))
    x = x / jnp.maximum(nrm, 1e-12)
    x_ref[...] = x
    y = x * dinv_ref[...]
    for q, ref in enumerate((y0_ref, y1_ref, y2_ref, y3_ref)):
        ref[...] = y[:, q * DIM_Q:(q + 1) * DIM_Q]


def _tc_pref(preference, dinv_users):
    g = NUM_USER // _BM
    qspec = pl.BlockSpec((_BM, DIM_Q), lambda i: (i, 0))
    return pl.pallas_call(
        _pref_body,
        grid=(g,),
        in_specs=[
            pl.BlockSpec((_BM, DIM), lambda i: (i, 0)),
            pl.BlockSpec((_BM, 1), lambda i: (i, 0)),
        ],
        out_specs=[pl.BlockSpec((_BM, DIM), lambda i: (i, 0))] + [qspec] * 4,
        out_shape=[jax.ShapeDtypeStruct((NUM_USER, DIM), jnp.float32)]
        + [jax.ShapeDtypeStruct((NUM_USER, DIM_Q), jnp.float32)] * 4,
    )(preference, dinv_users)


_BMB = 1568


def _y2_body(s0_ref, s1_ref, s2_ref, s3_ref, dinv2_ref,
             y0_ref, y1_ref, y2_ref, y3_ref):
    i = pl.program_id(0)
    rows = i * _BMB + lax.broadcasted_iota(jnp.int32, (_BMB, 1), 0)
    scale = jnp.where(rows < N_NODES, dinv2_ref[...], 0.0)
    for s_ref, y_ref in ((s0_ref, y0_ref), (s1_ref, y1_ref),
                         (s2_ref, y2_ref), (s3_ref, y3_ref)):
        y_ref[...] = s_ref[...] * scale


def _tc_y2(sq, dinv2):
    g = N_PAD // _BMB
    qspec = pl.BlockSpec((_BMB, DIM_Q), lambda i: (i, 0))
    return pl.pallas_call(
        _y2_body,
        grid=(g,),
        in_specs=[qspec] * 4 + [pl.BlockSpec((_BMB, 1), lambda i: (i, 0))],
        out_specs=[qspec] * 4,
        out_shape=[jax.ShapeDtypeStruct((N_PAD, DIM_Q), jnp.float32)] * 4,
    )(*sq, dinv2)


def _combine_body(x_ref, a0, a1, a2, a3, b0, b1, b2, b3, dinv_ref, out_ref):
    s = jnp.concatenate(
        [a[...] + b[...] for a, b in ((a0, b0), (a1, b1), (a2, b2), (a3, b3))],
        axis=1)
    out_ref[...] = x_ref[...] + dinv_ref[...] * s


def _tc_combine(x, s1q, s2q, dinv):
    g = N_NODES // _BM
    qspec = pl.BlockSpec((_BM, DIM_Q), lambda i: (i, 0))
    return pl.pallas_call(
        _combine_body,
        grid=(g,),
        in_specs=[pl.BlockSpec((_BM, DIM), lambda i: (i, 0))]
        + [qspec] * 8
        + [pl.BlockSpec((_BM, 1), lambda i: (i, 0))],
        out_specs=pl.BlockSpec((_BM, DIM), lambda i: (i, 0)),
        out_shape=jax.ShapeDtypeStruct((N_NODES, DIM), jnp.float32),
    )(x, *s1q, *s2q, dinv)


def kernel(edge_index, features, preference, W1, b1, W2, b2):
    row = edge_index[0].astype(jnp.int32)
    col = edge_index[1].astype(jnp.int32)
    padi = jnp.zeros((E_PAD - N_EDGES,), jnp.int32)
    row = jnp.concatenate([row, padi])
    col = jnp.concatenate([col, padi])

    zrow = jnp.zeros((ROWS_T,), jnp.float32)
    zblk = jnp.zeros((ROWS_T, DIM_Q), jnp.float32)

    rowp, degp0, degp1 = _sc_prep(row, col, zrow)
    dinv, dinv2 = _tc_degsum(degp0, degp1)
    dinv_col = dinv.reshape(N_PAD, 1)
    dinv2_col = dinv2.reshape(N_PAD, 1)

    x_items, *yq_i = _tc_items(features, W1, b1, W2, b2,
                               dinv_col[NUM_USER:N_NODES])
    x_pref, *yq_p = _tc_pref(preference, dinv_col[:NUM_USER])

    zpad = jnp.zeros((N_PAD - N_NODES, DIM_Q), jnp.float32)
    yq = [jnp.concatenate([p, i, zpad], axis=0) for p, i in zip(yq_p, yq_i)]

    s1q = _sc_layer(*yq, rowp, col, zblk)
    y2q = _tc_y2(s1q, dinv2_col)
    s2q = _sc_layer(*y2q, rowp, col, zblk)

    x = jnp.concatenate([x_pref, x_items], axis=0)
    x_hat = _tc_combine(x, s1q, s2q, dinv_col[:N_NODES])
    return (x_hat, preference)


# speedup vs baseline: 17.1202x; 3.4038x over previous
import functools

import jax
import jax.numpy as jnp
from jax import lax
from jax.experimental import pallas as pl
from jax.experimental.pallas import tpu as pltpu
from jax.experimental.pallas import tpu_sc as plsc

NUM_USER = 10000
NUM_ITEM = 40000
N_NODES = NUM_USER + NUM_ITEM
DIM = 64
DIM_H = 32
FEAT = 128
HID = 256

N_EDGES = 800000
CH = 128
NC, NS = 2, 16
E_PAD = 802816
PREP_CHUNKS = E_PAD // (CH * NC * NS)
LAYER_CHUNKS = E_PAD // (CH * NS)
E_ROWS = E_PAD // CH

N_PAD = 50176
ROWS_T = N_PAD // NS
ROWS_H = ROWS_T // 2

K_CH = 2
N_GROUPS = LAYER_CHUNKS // K_CH

_MESH = plsc.VectorSubcoreMesh(core_axis_name="c", subcore_axis_name="s",
                               num_cores=NC, num_subcores=NS)
_SC_PARAMS = pltpu.CompilerParams(use_tc_tiling_on_sc=False,
                                  needs_layout_passes=False)


@functools.partial(
    pl.kernel,
    out_type=[
        jax.ShapeDtypeStruct((E_PAD,), jnp.int32),
        jax.ShapeDtypeStruct((N_PAD,), jnp.float32),
        jax.ShapeDtypeStruct((N_PAD,), jnp.float32),
    ],
    mesh=_MESH,
    compiler_params=_SC_PARAMS,
    scratch_types=[
        pltpu.VMEM((CH,), jnp.int32),
        pltpu.VMEM((CH,), jnp.int32),
        pltpu.VMEM((CH,), jnp.int32),
        pltpu.VMEM((CH,), jnp.float32),
        pltpu.VMEM((ROWS_T // 4,), jnp.float32),
        pltpu.VMEM_SHARED((N_PAD,), jnp.float32),
    ],
)
def _sc_prep(row_hbm, col_hbm, rowp_hbm, degp0_hbm, degp1_hbm,
             rv, cv, rpv, ones_v, dbuf, acc1):
    cid = lax.axis_index("c")
    t = lax.axis_index("s")
    wid = cid * NS + t
    dummy = N_NODES + wid

    z16 = jnp.zeros((16,), jnp.float32)

    def zrow(i, _):
        dbuf[pl.ds(i * 16, 16)] = z16
        return ()

    lax.fori_loop(0, ROWS_T // 64, zrow, ())
    for q in range(4):
        pltpu.sync_copy(dbuf, acc1.at[pl.ds(t * ROWS_T + q * (ROWS_T // 4),
                                            ROWS_T // 4)])
    plsc.subcore_barrier()

    def chunk(j, _):
        base = (wid * PREP_CHUNKS + j) * CH
        pltpu.sync_copy(row_hbm.at[pl.ds(base, CH)], rv)
        pltpu.sync_copy(col_hbm.at[pl.ds(base, CH)], cv)
        for k in range(CH // 16):
            r = rv[pl.ds(k * 16, 16)]
            c = cv[pl.ds(k * 16, 16)]
            m = r != c
            rpv[pl.ds(k * 16, 16)] = jnp.where(m, r, dummy)
            ones_v[pl.ds(k * 16, 16)] = jnp.where(m, 1.0, 0.0)
        pltpu.sync_copy(rpv, rowp_hbm.at[pl.ds(base, CH)])
        pltpu.sync_copy(ones_v, acc1.at[rv], add=True)
        return ()

    lax.fori_loop(0, PREP_CHUNKS, chunk, ())
    plsc.subcore_barrier()

    for q in range(4):
        sl = pl.ds(t * ROWS_T + q * (ROWS_T // 4), ROWS_T // 4)
        pltpu.sync_copy(acc1.at[sl], dbuf)
        @pl.when(cid == 0)
        def _():
            pltpu.sync_copy(dbuf, degp0_hbm.at[sl])
        @pl.when(cid == 1)
        def _():
            pltpu.sync_copy(dbuf, degp1_hbm.at[sl])


CQ = ROWS_T // 14


@functools.partial(
    pl.kernel,
    out_type=[jax.ShapeDtypeStruct((N_PAD, DIM_H), jnp.float32)
              for _ in range(4)],
    mesh=_MESH,
    compiler_params=_SC_PARAMS,
    scratch_types=[
        pltpu.VMEM((2 * K_CH, CH), jnp.int32),
        pltpu.VMEM((2 * K_CH, CH), jnp.int32),
        pltpu.VMEM((2 * K_CH * CH, DIM_H), jnp.float32),
        pltpu.VMEM((CQ, DIM_H), jnp.float32),
        pltpu.VMEM((CQ,), jnp.float32),
        pltpu.VMEM_SHARED((N_PAD, DIM_H), jnp.float32),
        pltpu.SemaphoreType.DMA,
        pltpu.SemaphoreType.DMA,
        pltpu.SemaphoreType.DMA,
    ],
)
def _sc_layers(y0_hbm, y1_hbm, ei_hbm, d2_hbm,
               y2a_hbm, y2b_hbm, s0_hbm, s1_hbm,
               idx_r, idx_c, gbuf, wbuf, dbuf, acc, isem, gsem, ssem):
    cid = lax.axis_index("c")
    t = lax.axis_index("s")
    cbase = t * LAYER_CHUNKS

    def issue_idx(g, b):
        sl = pl.ds(b * K_CH, K_CH)
        pltpu.async_copy(ei_hbm.at[pl.ds(cbase + g * K_CH, K_CH)],
                         idx_r.at[sl], isem)
        pltpu.async_copy(ei_hbm.at[pl.ds(E_ROWS + cbase + g * K_CH, K_CH)],
                         idx_c.at[sl], isem)

    def wait_idx():
        sl = pl.ds(0, K_CH)
        pltpu.make_async_copy(ei_hbm.at[sl], idx_r.at[sl], isem).wait()
        pltpu.make_async_copy(ei_hbm.at[sl], idx_c.at[sl], isem).wait()

    def drain_scatters(yref):
        def w(j, _):
            pltpu.make_async_copy(gbuf.at[pl.ds(j * CH, CH)],
                                  acc.at[idx_c.at[j]], ssem).wait()
            return ()
        lax.fori_loop(0, K_CH, w, ())

    def do_group(g, b, yref):
        wait_idx()

        def fire(j, _):
            sl = b * K_CH + j
            pltpu.async_copy(yref.at[idx_r.at[sl]],
                             gbuf.at[pl.ds(sl * CH, CH)], gsem)
            return ()
        lax.fori_loop(0, K_CH, fire, ())

        @pl.when(g >= 1)
        def _():
            drain_scatters(yref)
        @pl.when(g + 1 < N_GROUPS)
        def _():
            issue_idx(g + 1, 1 - b)

        def gwait(j, _):
            pltpu.make_async_copy(yref.at[idx_r.at[0]],
                                  gbuf.at[pl.ds(0, CH)], gsem).wait()
            return ()
        lax.fori_loop(0, K_CH, gwait, ())

        def scat(j, _):
            sl = b * K_CH + j
            pltpu.async_copy(gbuf.at[pl.ds(sl * CH, CH)],
                             acc.at[idx_c.at[sl]], ssem, add=True)
            return ()
        lax.fori_loop(0, K_CH, scat, ())

    def run_pass(ya, yb):
        issue_idx(0, 0)

        def pair(i, _):
            @pl.when(cid == 0)
            def _():
                do_group(2 * i, 0, ya)
                do_group(2 * i + 1, 1, ya)
            @pl.when(cid == 1)
            def _():
                do_group(2 * i, 0, yb)
                do_group(2 * i + 1, 1, yb)
            return ()

        lax.fori_loop(0, N_GROUPS // 2, pair, ())
        drain_scatters(ya)

    z16 = jnp.zeros((16,), jnp.float32)

    def zrow(r, _):
        for c2 in range(DIM_H // 16):
            wbuf[r, pl.ds(c2 * 16, 16)] = z16
        return ()

    lax.fori_loop(0, CQ, zrow, ())
    for q in range(14):
        pltpu.sync_copy(wbuf, acc.at[pl.ds(t * ROWS_T + q * CQ, CQ)])
    plsc.subcore_barrier()

    run_pass(y0_hbm, y1_hbm)
    plsc.subcore_barrier()

    for q in range(14):
        rows = pl.ds(t * ROWS_T + q * CQ, CQ)
        pltpu.sync_copy(acc.at[rows], wbuf)
        pltpu.sync_copy(d2_hbm.at[rows], dbuf)

        def scale(r, _):
            b = plsc.load_gather(dbuf, [jnp.full((16,), r, jnp.int32)])
            for c2 in range(DIM_H // 16):
                sl = pl.ds(c2 * 16, 16)
                wbuf[r, sl] = wbuf[r, sl] * b
            return ()

        lax.fori_loop(0, CQ, scale, ())
        @pl.when(cid == 0)
        def _():
            pltpu.sync_copy(wbuf, y2a_hbm.at[rows])
        @pl.when(cid == 1)
        def _():
            pltpu.sync_copy(wbuf, y2b_hbm.at[rows])
    plsc.subcore_barrier()

    run_pass(y2a_hbm, y2b_hbm)
    plsc.subcore_barrier()

    for q in range(14):
        rows = pl.ds(t * ROWS_T + q * CQ, CQ)
        pltpu.sync_copy(acc.at[rows], wbuf)
        @pl.when(cid == 0)
        def _():
            pltpu.sync_copy(wbuf, s0_hbm.at[rows])
        @pl.when(cid == 1)
        def _():
            pltpu.sync_copy(wbuf, s1_hbm.at[rows])


def _degsum_body(p0_ref, p1_ref, dinv_ref, dinv2_ref):
    deg = p0_ref[...] + p1_ref[...]
    dinv_ref[...] = lax.rsqrt(deg)
    dinv2_ref[...] = 1.0 / deg


def _tc_degsum(degp0, degp1):
    return pl.pallas_call(
        _degsum_body,
        out_shape=[
            jax.ShapeDtypeStruct((N_PAD,), jnp.float32),
            jax.ShapeDtypeStruct((N_PAD,), jnp.float32),
        ],
    )(degp0, degp1)


_BM = 2000


def _mlp_body(feat_ref, w1_ref, b1_ref, w2_ref, b2_ref, dinv_ref,
              x_ref, y0_ref, y1_ref):
    t1 = lax.dot_general(feat_ref[...], w1_ref[...],
                         (((1,), (1,)), ((), ())),
                         preferred_element_type=jnp.float32)
    t1 = t1 + b1_ref[...]
    t1 = jnp.where(t1 >= 0, t1, 0.01 * t1)
    x = lax.dot_general(t1, w2_ref[...], (((1,), (1,)), ((), ())),
                        preferred_element_type=jnp.float32)
    x = x + b2_ref[...]
    nrm = jnp.sqrt(jnp.sum(x * x, axis=1, keepdims=True))
    x = x / jnp.maximum(nrm, 1e-12)
    x_ref[...] = x
    y = x * dinv_ref[...]
    y0_ref[...] = y[:, :DIM_H]
    y1_ref[...] = y[:, DIM_H:]


def _tc_items(features, W1, b1, W2, b2, dinv_items):
    g = NUM_ITEM // _BM
    hspec = pl.BlockSpec((_BM, DIM_H), lambda i: (i, 0))
    return pl.pallas_call(
        _mlp_body,
        grid=(g,),
        in_specs=[
            pl.BlockSpec((_BM, FEAT), lambda i: (i, 0)),
            pl.BlockSpec((HID, FEAT), lambda i: (0, 0)),
            pl.BlockSpec((1, HID), lambda i: (0, 0)),
            pl.BlockSpec((DIM, HID), lambda i: (0, 0)),
            pl.BlockSpec((1, DIM), lambda i: (0, 0)),
            pl.BlockSpec((_BM, 1), lambda i: (i, 0)),
        ],
        out_specs=[pl.BlockSpec((_BM, DIM), lambda i: (i, 0)), hspec, hspec],
        out_shape=[
            jax.ShapeDtypeStruct((NUM_ITEM, DIM), jnp.float32),
            jax.ShapeDtypeStruct((NUM_ITEM, DIM_H), jnp.float32),
            jax.ShapeDtypeStruct((NUM_ITEM, DIM_H), jnp.float32),
        ],
    )(features, W1, b1.reshape(1, HID), W2, b2.reshape(1, DIM), dinv_items)


def _pref_body(p_ref, dinv_ref, x_ref, y0_ref, y1_ref):
    x = p_ref[...]
    nrm = jnp.sqrt(jnp.sum(x * x, axis=1, keepdims=True))
    x = x / jnp.maximum(nrm, 1e-12)
    x_ref[...] = x
    y = x * dinv_ref[...]
    y0_ref[...] = y[:, :DIM_H]
    y1_ref[...] = y[:, DIM_H:]


def _tc_pref(preference, dinv_users):
    g = NUM_USER // _BM
    hspec = pl.BlockSpec((_BM, DIM_H), lambda i: (i, 0))
    return pl.pallas_call(
        _pref_body,
        grid=(g,),
        in_specs=[
            pl.BlockSpec((_BM, DIM), lambda i: (i, 0)),
            pl.BlockSpec((_BM, 1), lambda i: (i, 0)),
        ],
        out_specs=[pl.BlockSpec((_BM, DIM), lambda i: (i, 0)), hspec, hspec],
        out_shape=[
            jax.ShapeDtypeStruct((NUM_USER, DIM), jnp.float32),
            jax.ShapeDtypeStruct((NUM_USER, DIM_H), jnp.float32),
            jax.ShapeDtypeStruct((NUM_USER, DIM_H), jnp.float32),
        ],
    )(preference, dinv_users)


def _combine_body(x_ref, a0, a1, dinv_ref, out_ref):
    s = jnp.concatenate([a0[...], a1[...]], axis=1)
    out_ref[...] = x_ref[...] + dinv_ref[...] * s


def _tc_combine(x, sh, dinv):
    g = N_NODES // _BM
    hspec = pl.BlockSpec((_BM, DIM_H), lambda i: (i, 0))
    return pl.pallas_call(
        _combine_body,
        grid=(g,),
        in_specs=[pl.BlockSpec((_BM, DIM), lambda i: (i, 0))]
        + [hspec] * 2
        + [pl.BlockSpec((_BM, 1), lambda i: (i, 0))],
        out_specs=pl.BlockSpec((_BM, DIM), lambda i: (i, 0)),
        out_shape=jax.ShapeDtypeStruct((N_NODES, DIM), jnp.float32),
    )(x, *sh, dinv)


def kernel(edge_index, features, preference, W1, b1, W2, b2):
    row = edge_index[0].astype(jnp.int32)
    col = edge_index[1].astype(jnp.int32)
    padi = jnp.zeros((E_PAD - N_EDGES,), jnp.int32)
    row = jnp.concatenate([row, padi])
    col = jnp.concatenate([col, padi])

    rowp, degp0, degp1 = _sc_prep(row, col)
    ei = jnp.concatenate([rowp, col]).reshape(2 * E_ROWS, CH)

    dinv, dinv2 = _tc_degsum(degp0, degp1)
    dinv_col = dinv.reshape(N_PAD, 1)
    d2 = jnp.where(jnp.arange(N_PAD) < N_NODES, dinv2, 0.0)

    x_items, y0_i, y1_i = _tc_items(features, W1, b1, W2, b2,
                                    dinv_col[NUM_USER:N_NODES])
    x_pref, y0_p, y1_p = _tc_pref(preference, dinv_col[:NUM_USER])

    zpad = jnp.zeros((N_PAD - N_NODES, DIM_H), jnp.float32)
    y0 = jnp.concatenate([y0_p, y0_i, zpad], axis=0)
    y1 = jnp.concatenate([y1_p, y1_i, zpad], axis=0)

    _, _, st0, st1 = _sc_layers(y0, y1, ei, d2)

    x = jnp.concatenate([x_pref, x_items], axis=0)
    x_hat = _tc_combine(x, (st0[:N_NODES], st1[:N_NODES]), dinv_col[:N_NODES])
    return (x_hat, preference)


# speedup vs baseline: 19.9756x; 1.1668x over previous
import functools

import jax
import jax.numpy as jnp
from jax import lax
from jax.experimental import pallas as pl
from jax.experimental.pallas import tpu as pltpu
from jax.experimental.pallas import tpu_sc as plsc

NUM_USER = 10000
NUM_ITEM = 40000
N_NODES = NUM_USER + NUM_ITEM
DIM = 64
DIM_H = 32
FEAT = 128
HID = 256

N_EDGES = 800000
CH = 128
NC, NS = 2, 16
E_PAD = 802816
PREP_CHUNKS = E_PAD // (CH * NC * NS)
LAYER_CHUNKS = E_PAD // (CH * NS)
E_ROWS = E_PAD // CH

N_PAD = 50176
ROWS_T = N_PAD // NS
ROWS_H = ROWS_T // 2

K_CH = 2
N_GROUPS = LAYER_CHUNKS // K_CH

_MESH = plsc.VectorSubcoreMesh(core_axis_name="c", subcore_axis_name="s",
                               num_cores=NC, num_subcores=NS)
_SC_PARAMS = pltpu.CompilerParams(use_tc_tiling_on_sc=False,
                                  needs_layout_passes=False)


PREP_ROWS = E_ROWS // (NC * NS)
RG = 2
PREP_GROUPS = PREP_ROWS // RG


@functools.partial(
    pl.kernel,
    out_type=[
        jax.ShapeDtypeStruct((E_ROWS, CH), jnp.int32),
        jax.ShapeDtypeStruct((N_PAD,), jnp.float32),
        jax.ShapeDtypeStruct((N_PAD,), jnp.float32),
    ],
    mesh=_MESH,
    compiler_params=_SC_PARAMS,
    scratch_types=[
        pltpu.VMEM((2 * RG, CH), jnp.int32),
        pltpu.VMEM((2 * RG, CH), jnp.int32),
        pltpu.VMEM((2 * RG, CH), jnp.int32),
        pltpu.VMEM((2 * RG, CH), jnp.float32),
        pltpu.VMEM((ROWS_T // 4,), jnp.float32),
        pltpu.VMEM_SHARED((N_PAD,), jnp.float32),
        pltpu.SemaphoreType.DMA,
        pltpu.SemaphoreType.DMA,
    ],
)
def _sc_prep(row_hbm, col_hbm, rowp_hbm, degp0_hbm, degp1_hbm,
             rv, cv, rpv, ones_v, dbuf, acc1, isem, wsem):
    cid = lax.axis_index("c")
    t = lax.axis_index("s")
    wid = cid * NS + t
    dummy = N_NODES + wid
    rbase = wid * PREP_ROWS

    z16 = jnp.zeros((16,), jnp.float32)

    def zrow(i, _):
        dbuf[pl.ds(i * 16, 16)] = z16
        return ()

    lax.fori_loop(0, ROWS_T // 64, zrow, ())
    for q in range(4):
        pltpu.sync_copy(dbuf, acc1.at[pl.ds(t * ROWS_T + q * (ROWS_T // 4),
                                            ROWS_T // 4)])
    plsc.subcore_barrier()

    def issue(g, b):
        sl = pl.ds(b * RG, RG)
        rows = pl.ds(rbase + g * RG, RG)
        pltpu.async_copy(row_hbm.at[rows], rv.at[sl], isem)
        pltpu.async_copy(col_hbm.at[rows], cv.at[sl], isem)

    def proc(g, b):
        sl = pl.ds(b * RG, RG)
        @pl.when(g >= 2)
        def _():
            pltpu.make_async_copy(rpv.at[sl], rowp_hbm.at[sl], wsem).wait()
        pltpu.make_async_copy(row_hbm.at[sl], rv.at[sl], isem).wait()
        pltpu.make_async_copy(col_hbm.at[sl], cv.at[sl], isem).wait()
        @pl.when(g + 1 < PREP_GROUPS)
        def _():
            issue(g + 1, 1 - b)
        for rr in range(RG):
            for k in range(CH // 16):
                ks = pl.ds(k * 16, 16)
                r = rv[b * RG + rr, ks]
                c = cv[b * RG + rr, ks]
                m = r != c
                rpv[b * RG + rr, ks] = jnp.where(m, r, dummy)
                ones_v[b * RG + rr, ks] = jnp.where(m, 1.0, 0.0)
        pltpu.async_copy(rpv.at[sl], rowp_hbm.at[pl.ds(rbase + g * RG, RG)],
                         wsem)
        for rr in range(RG):
            pltpu.sync_copy(ones_v.at[b * RG + rr],
                            acc1.at[rv.at[b * RG + rr]], add=True)

    issue(0, 0)

    def pair(i, _):
        proc(2 * i, 0)
        proc(2 * i + 1, 1)
        return ()

    lax.fori_loop(0, PREP_GROUPS // 2, pair, ())
    for _ in range(2):
        pltpu.make_async_copy(rpv.at[pl.ds(0, RG)],
                              rowp_hbm.at[pl.ds(0, RG)], wsem).wait()
    plsc.subcore_barrier()

    for q in range(4):
        sl = pl.ds(t * ROWS_T + q * (ROWS_T // 4), ROWS_T // 4)
        pltpu.sync_copy(acc1.at[sl], dbuf)
        @pl.when(cid == 0)
        def _():
            pltpu.sync_copy(dbuf, degp0_hbm.at[sl])
        @pl.when(cid == 1)
        def _():
            pltpu.sync_copy(dbuf, degp1_hbm.at[sl])


CQ = ROWS_T // 14


@functools.partial(
    pl.kernel,
    out_type=[jax.ShapeDtypeStruct((N_PAD, DIM_H), jnp.float32)
              for _ in range(4)],
    mesh=_MESH,
    compiler_params=_SC_PARAMS,
    scratch_types=[
        pltpu.VMEM((2 * K_CH, CH), jnp.int32),
        pltpu.VMEM((2 * K_CH, CH), jnp.int32),
        pltpu.VMEM((2 * K_CH * CH, DIM_H), jnp.float32),
        pltpu.VMEM((CQ, DIM_H), jnp.float32),
        pltpu.VMEM((CQ,), jnp.float32),
        pltpu.VMEM_SHARED((N_PAD, DIM_H), jnp.float32),
        pltpu.SemaphoreType.DMA,
        pltpu.SemaphoreType.DMA,
        pltpu.SemaphoreType.DMA,
    ],
)
def _sc_layers(y0_hbm, y1_hbm, ei_hbm, d2_hbm,
               y2a_hbm, y2b_hbm, s0_hbm, s1_hbm,
               idx_r, idx_c, gbuf, wbuf, dbuf, acc, isem, gsem, ssem):
    cid = lax.axis_index("c")
    t = lax.axis_index("s")
    cbase = t * LAYER_CHUNKS

    def issue_idx(g, b):
        sl = pl.ds(b * K_CH, K_CH)
        pltpu.async_copy(ei_hbm.at[pl.ds(cbase + g * K_CH, K_CH)],
                         idx_r.at[sl], isem)
        pltpu.async_copy(ei_hbm.at[pl.ds(E_ROWS + cbase + g * K_CH, K_CH)],
                         idx_c.at[sl], isem)

    def wait_idx():
        sl = pl.ds(0, K_CH)
        pltpu.make_async_copy(ei_hbm.at[sl], idx_r.at[sl], isem).wait()
        pltpu.make_async_copy(ei_hbm.at[sl], idx_c.at[sl], isem).wait()

    def drain_scatters(yref):
        def w(j, _):
            pltpu.make_async_copy(gbuf.at[pl.ds(j * CH, CH)],
                                  acc.at[idx_c.at[j]], ssem).wait()
            return ()
        lax.fori_loop(0, K_CH, w, ())

    def do_group(g, b, yref):
        wait_idx()

        def fire(j, _):
            sl = b * K_CH + j
            pltpu.async_copy(yref.at[idx_r.at[sl]],
                             gbuf.at[pl.ds(sl * CH, CH)], gsem)
            return ()
        lax.fori_loop(0, K_CH, fire, ())

        @pl.when(g >= 1)
        def _():
            drain_scatters(yref)
        @pl.when(g + 1 < N_GROUPS)
        def _():
            issue_idx(g + 1, 1 - b)

        def gwait(j, _):
            pltpu.make_async_copy(yref.at[idx_r.at[0]],
                                  gbuf.at[pl.ds(0, CH)], gsem).wait()
            return ()
        lax.fori_loop(0, K_CH, gwait, ())

        def scat(j, _):
            sl = b * K_CH + j
            pltpu.async_copy(gbuf.at[pl.ds(sl * CH, CH)],
                             acc.at[idx_c.at[sl]], ssem, add=True)
            return ()
        lax.fori_loop(0, K_CH, scat, ())

    def run_pass(ya, yb):
        issue_idx(0, 0)

        def pair(i, _):
            @pl.when(cid == 0)
            def _():
                do_group(2 * i, 0, ya)
                do_group(2 * i + 1, 1, ya)
            @pl.when(cid == 1)
            def _():
                do_group(2 * i, 0, yb)
                do_group(2 * i + 1, 1, yb)
            return ()

        lax.fori_loop(0, N_GROUPS // 2, pair, ())
        drain_scatters(ya)

    z16 = jnp.zeros((16,), jnp.float32)

    def zrow(r, _):
        for c2 in range(DIM_H // 16):
            wbuf[r, pl.ds(c2 * 16, 16)] = z16
        return ()

    lax.fori_loop(0, CQ, zrow, ())
    for q in range(14):
        pltpu.sync_copy(wbuf, acc.at[pl.ds(t * ROWS_T + q * CQ, CQ)])
    plsc.subcore_barrier()

    run_pass(y0_hbm, y1_hbm)
    plsc.subcore_barrier()

    for q in range(14):
        rows = pl.ds(t * ROWS_T + q * CQ, CQ)
        pltpu.sync_copy(acc.at[rows], wbuf)
        pltpu.sync_copy(d2_hbm.at[rows], dbuf)

        def scale(r, _):
            b = plsc.load_gather(dbuf, [jnp.full((16,), r, jnp.int32)])
            for c2 in range(DIM_H // 16):
                sl = pl.ds(c2 * 16, 16)
                wbuf[r, sl] = wbuf[r, sl] * b
            return ()

        lax.fori_loop(0, CQ, scale, ())
        @pl.when(cid == 0)
        def _():
            pltpu.sync_copy(wbuf, y2a_hbm.at[rows])
        @pl.when(cid == 1)
        def _():
            pltpu.sync_copy(wbuf, y2b_hbm.at[rows])
    plsc.subcore_barrier()

    run_pass(y2a_hbm, y2b_hbm)
    plsc.subcore_barrier()

    for q in range(14):
        rows = pl.ds(t * ROWS_T + q * CQ, CQ)
        pltpu.sync_copy(acc.at[rows], wbuf)
        @pl.when(cid == 0)
        def _():
            pltpu.sync_copy(wbuf, s0_hbm.at[rows])
        @pl.when(cid == 1)
        def _():
            pltpu.sync_copy(wbuf, s1_hbm.at[rows])


def _degsum_body(p0_ref, p1_ref, dinv_ref, dinv2_ref):
    deg = p0_ref[...] + p1_ref[...]
    dinv_ref[...] = lax.rsqrt(deg)
    dinv2_ref[...] = 1.0 / deg


def _tc_degsum(degp0, degp1):
    return pl.pallas_call(
        _degsum_body,
        out_shape=[
            jax.ShapeDtypeStruct((N_PAD,), jnp.float32),
            jax.ShapeDtypeStruct((N_PAD,), jnp.float32),
        ],
    )(degp0, degp1)


_BM = 2000


def _mlp_body(feat_ref, w1_ref, b1_ref, w2_ref, b2_ref, dinv_ref,
              x_ref, y0_ref, y1_ref):
    t1 = lax.dot_general(feat_ref[...], w1_ref[...],
                         (((1,), (1,)), ((), ())),
                         preferred_element_type=jnp.float32)
    t1 = t1 + b1_ref[...]
    t1 = jnp.where(t1 >= 0, t1, 0.01 * t1)
    x = lax.dot_general(t1, w2_ref[...], (((1,), (1,)), ((), ())),
                        preferred_element_type=jnp.float32)
    x = x + b2_ref[...]
    nrm = jnp.sqrt(jnp.sum(x * x, axis=1, keepdims=True))
    x = x / jnp.maximum(nrm, 1e-12)
    x_ref[...] = x
    y = x * dinv_ref[...]
    y0_ref[...] = y[:, :DIM_H]
    y1_ref[...] = y[:, DIM_H:]


def _tc_items(features, W1, b1, W2, b2, dinv_items):
    g = NUM_ITEM // _BM
    hspec = pl.BlockSpec((_BM, DIM_H), lambda i: (i, 0))
    return pl.pallas_call(
        _mlp_body,
        grid=(g,),
        in_specs=[
            pl.BlockSpec((_BM, FEAT), lambda i: (i, 0)),
            pl.BlockSpec((HID, FEAT), lambda i: (0, 0)),
            pl.BlockSpec((1, HID), lambda i: (0, 0)),
            pl.BlockSpec((DIM, HID), lambda i: (0, 0)),
            pl.BlockSpec((1, DIM), lambda i: (0, 0)),
            pl.BlockSpec((_BM, 1), lambda i: (i, 0)),
        ],
        out_specs=[pl.BlockSpec((_BM, DIM), lambda i: (i, 0)), hspec, hspec],
        out_shape=[
            jax.ShapeDtypeStruct((NUM_ITEM, DIM), jnp.float32),
            jax.ShapeDtypeStruct((NUM_ITEM, DIM_H), jnp.float32),
            jax.ShapeDtypeStruct((NUM_ITEM, DIM_H), jnp.float32),
        ],
    )(features, W1, b1.reshape(1, HID), W2, b2.reshape(1, DIM), dinv_items)


def _pref_body(p_ref, dinv_ref, x_ref, y0_ref, y1_ref):
    x = p_ref[...]
    nrm = jnp.sqrt(jnp.sum(x * x, axis=1, keepdims=True))
    x = x / jnp.maximum(nrm, 1e-12)
    x_ref[...] = x
    y = x * dinv_ref[...]
    y0_ref[...] = y[:, :DIM_H]
    y1_ref[...] = y[:, DIM_H:]


def _tc_pref(preference, dinv_users):
    g = NUM_USER // _BM
    hspec = pl.BlockSpec((_BM, DIM_H), lambda i: (i, 0))
    return pl.pallas_call(
        _pref_body,
        grid=(g,),
        in_specs=[
            pl.BlockSpec((_BM, DIM), lambda i: (i, 0)),
            pl.BlockSpec((_BM, 1), lambda i: (i, 0)),
        ],
        out_specs=[pl.BlockSpec((_BM, DIM), lambda i: (i, 0)), hspec, hspec],
        out_shape=[
            jax.ShapeDtypeStruct((NUM_USER, DIM), jnp.float32),
            jax.ShapeDtypeStruct((NUM_USER, DIM_H), jnp.float32),
            jax.ShapeDtypeStruct((NUM_USER, DIM_H), jnp.float32),
        ],
    )(preference, dinv_users)


def _combine_body(x_ref, a0, a1, dinv_ref, out_ref):
    s = jnp.concatenate([a0[...], a1[...]], axis=1)
    out_ref[...] = x_ref[...] + dinv_ref[...] * s


def _tc_combine(x, sh, dinv):
    g = N_NODES // _BM
    hspec = pl.BlockSpec((_BM, DIM_H), lambda i: (i, 0))
    return pl.pallas_call(
        _combine_body,
        grid=(g,),
        in_specs=[pl.BlockSpec((_BM, DIM), lambda i: (i, 0))]
        + [hspec] * 2
        + [pl.BlockSpec((_BM, 1), lambda i: (i, 0))],
        out_specs=pl.BlockSpec((_BM, DIM), lambda i: (i, 0)),
        out_shape=jax.ShapeDtypeStruct((N_NODES, DIM), jnp.float32),
    )(x, *sh, dinv)


def kernel(edge_index, features, preference, W1, b1, W2, b2):
    row = edge_index[0].astype(jnp.int32)
    col = edge_index[1].astype(jnp.int32)
    padi = jnp.zeros((E_PAD - N_EDGES,), jnp.int32)
    row = jnp.concatenate([row, padi])
    col = jnp.concatenate([col, padi])

    rowp, degp0, degp1 = _sc_prep(row.reshape(E_ROWS, CH),
                                  col.reshape(E_ROWS, CH))
    ei = jnp.concatenate([rowp.reshape(-1), col]).reshape(2 * E_ROWS, CH)

    dinv, dinv2 = _tc_degsum(degp0, degp1)
    dinv_col = dinv.reshape(N_PAD, 1)
    d2 = jnp.where(jnp.arange(N_PAD) < N_NODES, dinv2, 0.0)

    x_items, y0_i, y1_i = _tc_items(features, W1, b1, W2, b2,
                                    dinv_col[NUM_USER:N_NODES])
    x_pref, y0_p, y1_p = _tc_pref(preference, dinv_col[:NUM_USER])

    zpad = jnp.zeros((N_PAD - N_NODES, DIM_H), jnp.float32)
    y0 = jnp.concatenate([y0_p, y0_i, zpad], axis=0)
    y1 = jnp.concatenate([y1_p, y1_i, zpad], axis=0)

    _, _, st0, st1 = _sc_layers(y0, y1, ei, d2)

    x = jnp.concatenate([x_pref, x_items], axis=0)
    x_hat = _tc_combine(x, (st0[:N_NODES], st1[:N_NODES]), dinv_col[:N_NODES])
    return (x_hat, preference)


# speedup vs baseline: 21.1944x; 1.0610x over previous
import functools

import jax
import jax.numpy as jnp
from jax import lax
from jax.experimental import pallas as pl
from jax.experimental.pallas import tpu as pltpu
from jax.experimental.pallas import tpu_sc as plsc

NUM_USER = 10000
NUM_ITEM = 40000
N_NODES = NUM_USER + NUM_ITEM
DIM = 64
DIM_H = 32
FEAT = 128
HID = 256

N_EDGES = 800000
CH = 128
NC, NS = 2, 16
E_PAD = 802816
PREP_CHUNKS = E_PAD // (CH * NC * NS)
LAYER_CHUNKS = E_PAD // (CH * NS)
E_ROWS = E_PAD // CH

N_PAD = 50176
ROWS_T = N_PAD // NS
ROWS_H = ROWS_T // 2

K_CH = 2
N_GROUPS = LAYER_CHUNKS // K_CH

_MESH = plsc.VectorSubcoreMesh(core_axis_name="c", subcore_axis_name="s",
                               num_cores=NC, num_subcores=NS)
_SC_PARAMS = pltpu.CompilerParams(use_tc_tiling_on_sc=False,
                                  needs_layout_passes=False)


PREP_ROWS = E_ROWS // (NC * NS)
RG = 2
PREP_GROUPS = PREP_ROWS // RG


@functools.partial(
    pl.kernel,
    out_type=[
        jax.ShapeDtypeStruct((E_ROWS, CH), jnp.int32),
        jax.ShapeDtypeStruct((N_PAD,), jnp.float32),
        jax.ShapeDtypeStruct((N_PAD,), jnp.float32),
    ],
    mesh=_MESH,
    compiler_params=_SC_PARAMS,
    scratch_types=[
        pltpu.VMEM((2 * RG, CH), jnp.int32),
        pltpu.VMEM((2 * RG, CH), jnp.int32),
        pltpu.VMEM((2 * RG, CH), jnp.int32),
        pltpu.VMEM((2 * RG, CH), jnp.float32),
        pltpu.VMEM((ROWS_T // 4,), jnp.float32),
        pltpu.VMEM_SHARED((N_PAD,), jnp.float32),
        pltpu.SemaphoreType.DMA,
        pltpu.SemaphoreType.DMA,
    ],
)
def _sc_prep(row_hbm, col_hbm, rowp_hbm, degp0_hbm, degp1_hbm,
             rv, cv, rpv, ones_v, dbuf, acc1, isem, wsem):
    cid = lax.axis_index("c")
    t = lax.axis_index("s")
    wid = cid * NS + t
    dummy = N_NODES + wid
    rbase = wid * PREP_ROWS

    z16 = jnp.zeros((16,), jnp.float32)

    def zrow(i, _):
        dbuf[pl.ds(i * 16, 16)] = z16
        return ()

    lax.fori_loop(0, ROWS_T // 64, zrow, ())
    for q in range(4):
        pltpu.sync_copy(dbuf, acc1.at[pl.ds(t * ROWS_T + q * (ROWS_T // 4),
                                            ROWS_T // 4)])
    plsc.subcore_barrier()

    def issue(g, b):
        sl = pl.ds(b * RG, RG)
        rows = pl.ds(rbase + g * RG, RG)
        pltpu.async_copy(row_hbm.at[rows], rv.at[sl], isem)
        pltpu.async_copy(col_hbm.at[rows], cv.at[sl], isem)

    def proc(g, b):
        sl = pl.ds(b * RG, RG)
        @pl.when(g >= 2)
        def _():
            pltpu.make_async_copy(rpv.at[sl], rowp_hbm.at[sl], wsem).wait()
        pltpu.make_async_copy(row_hbm.at[sl], rv.at[sl], isem).wait()
        pltpu.make_async_copy(col_hbm.at[sl], cv.at[sl], isem).wait()
        @pl.when(g + 1 < PREP_GROUPS)
        def _():
            issue(g + 1, 1 - b)
        for rr in range(RG):
            for k in range(CH // 16):
                ks = pl.ds(k * 16, 16)
                r = rv[b * RG + rr, ks]
                c = cv[b * RG + rr, ks]
                m = r != c
                rpv[b * RG + rr, ks] = jnp.where(m, r, dummy)
                ones_v[b * RG + rr, ks] = jnp.where(m, 1.0, 0.0)
        pltpu.async_copy(rpv.at[sl], rowp_hbm.at[pl.ds(rbase + g * RG, RG)],
                         wsem)
        for rr in range(RG):
            pltpu.sync_copy(ones_v.at[b * RG + rr],
                            acc1.at[rv.at[b * RG + rr]], add=True)

    issue(0, 0)

    def pair(i, _):
        proc(2 * i, 0)
        proc(2 * i + 1, 1)
        return ()

    lax.fori_loop(0, PREP_GROUPS // 2, pair, ())
    for _ in range(2):
        pltpu.make_async_copy(rpv.at[pl.ds(0, RG)],
                              rowp_hbm.at[pl.ds(0, RG)], wsem).wait()
    plsc.subcore_barrier()

    for q in range(4):
        sl = pl.ds(t * ROWS_T + q * (ROWS_T // 4), ROWS_T // 4)
        pltpu.sync_copy(acc1.at[sl], dbuf)
        @pl.when(cid == 0)
        def _():
            pltpu.sync_copy(dbuf, degp0_hbm.at[sl])
        @pl.when(cid == 1)
        def _():
            pltpu.sync_copy(dbuf, degp1_hbm.at[sl])


CQ = ROWS_T // 14


@functools.partial(
    pl.kernel,
    out_type=[jax.ShapeDtypeStruct((N_PAD, DIM_H), jnp.float32)
              for _ in range(4)],
    mesh=_MESH,
    compiler_params=_SC_PARAMS,
    scratch_types=[
        pltpu.VMEM((2 * K_CH, CH), jnp.int32),
        pltpu.VMEM((2 * K_CH, CH), jnp.int32),
        pltpu.VMEM((2 * K_CH * CH, DIM_H), jnp.float32),
        pltpu.VMEM((CQ, DIM_H), jnp.float32),
        pltpu.VMEM((CQ,), jnp.float32),
        pltpu.VMEM_SHARED((N_PAD, DIM_H), jnp.float32),
        pltpu.SemaphoreType.DMA,
        pltpu.SemaphoreType.DMA,
        pltpu.SemaphoreType.DMA,
    ],
)
def _sc_layers(y0_hbm, y1_hbm, ei_hbm, d2_hbm,
               y2a_hbm, y2b_hbm, s0_hbm, s1_hbm,
               idx_r, idx_c, gbuf, wbuf, dbuf, acc, isem, gsem, ssem):
    cid = lax.axis_index("c")
    t = lax.axis_index("s")
    cbase = t * LAYER_CHUNKS

    def issue_idx(g, b):
        sl = pl.ds(b * K_CH, K_CH)
        pltpu.async_copy(ei_hbm.at[pl.ds(cbase + g * K_CH, K_CH)],
                         idx_r.at[sl], isem)
        pltpu.async_copy(ei_hbm.at[pl.ds(E_ROWS + cbase + g * K_CH, K_CH)],
                         idx_c.at[sl], isem)

    def wait_idx():
        sl = pl.ds(0, K_CH)
        pltpu.make_async_copy(ei_hbm.at[sl], idx_r.at[sl], isem).wait()
        pltpu.make_async_copy(ei_hbm.at[sl], idx_c.at[sl], isem).wait()

    def drain_scatters(yref):
        def w(j, _):
            pltpu.make_async_copy(gbuf.at[pl.ds(j * CH, CH)],
                                  acc.at[idx_c.at[j]], ssem).wait()
            return ()
        lax.fori_loop(0, K_CH, w, ())

    def do_group(g, b, yref):
        wait_idx()

        def fire(j, _):
            sl = b * K_CH + j
            pltpu.async_copy(yref.at[idx_r.at[sl]],
                             gbuf.at[pl.ds(sl * CH, CH)], gsem)
            return ()
        lax.fori_loop(0, K_CH, fire, ())

        @pl.when(g >= 1)
        def _():
            drain_scatters(yref)
        @pl.when(g + 1 < N_GROUPS)
        def _():
            issue_idx(g + 1, 1 - b)

        def gwait(j, _):
            pltpu.make_async_copy(yref.at[idx_r.at[0]],
                                  gbuf.at[pl.ds(0, CH)], gsem).wait()
            return ()
        lax.fori_loop(0, K_CH, gwait, ())

        def scat(j, _):
            sl = b * K_CH + j
            pltpu.async_copy(gbuf.at[pl.ds(sl * CH, CH)],
                             acc.at[idx_c.at[sl]], ssem, add=True)
            return ()
        lax.fori_loop(0, K_CH, scat, ())

    def run_pass(ya, yb):
        issue_idx(0, 0)

        def pair(i, _):
            @pl.when(cid == 0)
            def _():
                do_group(2 * i, 0, ya)
                do_group(2 * i + 1, 1, ya)
            @pl.when(cid == 1)
            def _():
                do_group(2 * i, 0, yb)
                do_group(2 * i + 1, 1, yb)
            return ()

        lax.fori_loop(0, N_GROUPS // 2, pair, ())
        drain_scatters(ya)

    z16 = jnp.zeros((16,), jnp.float32)

    def zrow(r, _):
        for c2 in range(DIM_H // 16):
            wbuf[r, pl.ds(c2 * 16, 16)] = z16
        return ()

    lax.fori_loop(0, CQ, zrow, ())
    for q in range(14):
        pltpu.sync_copy(wbuf, acc.at[pl.ds(t * ROWS_T + q * CQ, CQ)])
    plsc.subcore_barrier()

    run_pass(y0_hbm, y1_hbm)
    plsc.subcore_barrier()

    for q in range(14):
        rows = pl.ds(t * ROWS_T + q * CQ, CQ)
        pltpu.sync_copy(acc.at[rows], wbuf)
        pltpu.sync_copy(d2_hbm.at[rows], dbuf)

        def scale(r, _):
            b = plsc.load_gather(dbuf, [jnp.full((16,), r, jnp.int32)])
            for c2 in range(DIM_H // 16):
                sl = pl.ds(c2 * 16, 16)
                wbuf[r, sl] = wbuf[r, sl] * b
            return ()

        lax.fori_loop(0, CQ, scale, ())
        @pl.when(cid == 0)
        def _():
            pltpu.sync_copy(wbuf, y2a_hbm.at[rows])
        @pl.when(cid == 1)
        def _():
            pltpu.sync_copy(wbuf, y2b_hbm.at[rows])
    plsc.subcore_barrier()

    run_pass(y2a_hbm, y2b_hbm)
    plsc.subcore_barrier()

    for q in range(14):
        rows = pl.ds(t * ROWS_T + q * CQ, CQ)
        pltpu.sync_copy(acc.at[rows], wbuf)
        @pl.when(cid == 0)
        def _():
            pltpu.sync_copy(wbuf, s0_hbm.at[rows])
        @pl.when(cid == 1)
        def _():
            pltpu.sync_copy(wbuf, s1_hbm.at[rows])


_BD = 3136


def _degscale_body(p0_ref, p1_ref, x_ref, dinv_ref, d2_ref, y0_ref, y1_ref):
    pid = pl.program_id(0)
    deg = p0_ref[...] + p1_ref[...]
    rows = pid * _BD + lax.broadcasted_iota(jnp.int32, (_BD, 1), 0)
    real = rows < N_NODES
    dinv = jnp.where(real, lax.rsqrt(deg), 0.0)
    dinv_ref[...] = dinv
    d2_ref[...] = jnp.where(real, 1.0 / deg, 0.0)
    y = x_ref[...] * dinv
    y0_ref[...] = y[:, :DIM_H]
    y1_ref[...] = y[:, DIM_H:]


def _tc_degscale(degp0, degp1, xp):
    cspec = pl.BlockSpec((_BD, 1), lambda i: (i, 0))
    hspec = pl.BlockSpec((_BD, DIM_H), lambda i: (i, 0))
    return pl.pallas_call(
        _degscale_body,
        grid=(N_PAD // _BD,),
        in_specs=[cspec, cspec, pl.BlockSpec((_BD, DIM), lambda i: (i, 0))],
        out_specs=[cspec, cspec, hspec, hspec],
        out_shape=[
            jax.ShapeDtypeStruct((N_PAD, 1), jnp.float32),
            jax.ShapeDtypeStruct((N_PAD, 1), jnp.float32),
            jax.ShapeDtypeStruct((N_PAD, DIM_H), jnp.float32),
            jax.ShapeDtypeStruct((N_PAD, DIM_H), jnp.float32),
        ],
    )(degp0.reshape(N_PAD, 1), degp1.reshape(N_PAD, 1), xp)


_BM = 2000


def _mlp_body(feat_ref, w1_ref, b1_ref, w2_ref, b2_ref, x_ref):
    t1 = lax.dot_general(feat_ref[...], w1_ref[...],
                         (((1,), (1,)), ((), ())),
                         preferred_element_type=jnp.float32)
    t1 = t1 + b1_ref[...]
    t1 = jnp.where(t1 >= 0, t1, 0.01 * t1)
    x = lax.dot_general(t1, w2_ref[...], (((1,), (1,)), ((), ())),
                        preferred_element_type=jnp.float32)
    x = x + b2_ref[...]
    nrm = jnp.sqrt(jnp.sum(x * x, axis=1, keepdims=True))
    x_ref[...] = x / jnp.maximum(nrm, 1e-12)


def _tc_items(features, W1, b1, W2, b2):
    g = NUM_ITEM // _BM
    return pl.pallas_call(
        _mlp_body,
        grid=(g,),
        in_specs=[
            pl.BlockSpec((_BM, FEAT), lambda i: (i, 0)),
            pl.BlockSpec((HID, FEAT), lambda i: (0, 0)),
            pl.BlockSpec((1, HID), lambda i: (0, 0)),
            pl.BlockSpec((DIM, HID), lambda i: (0, 0)),
            pl.BlockSpec((1, DIM), lambda i: (0, 0)),
        ],
        out_specs=pl.BlockSpec((_BM, DIM), lambda i: (i, 0)),
        out_shape=jax.ShapeDtypeStruct((NUM_ITEM, DIM), jnp.float32),
    )(features, W1, b1.reshape(1, HID), W2, b2.reshape(1, DIM))


def _pref_body(p_ref, x_ref):
    x = p_ref[...]
    nrm = jnp.sqrt(jnp.sum(x * x, axis=1, keepdims=True))
    x_ref[...] = x / jnp.maximum(nrm, 1e-12)


def _tc_pref(preference):
    g = NUM_USER // _BM
    return pl.pallas_call(
        _pref_body,
        grid=(g,),
        in_specs=[pl.BlockSpec((_BM, DIM), lambda i: (i, 0))],
        out_specs=pl.BlockSpec((_BM, DIM), lambda i: (i, 0)),
        out_shape=jax.ShapeDtypeStruct((NUM_USER, DIM), jnp.float32),
    )(preference)


def _combine_body(x_ref, a0, a1, dinv_ref, out_ref):
    s = jnp.concatenate([a0[...], a1[...]], axis=1)
    out_ref[...] = x_ref[...] + dinv_ref[...] * s


def _tc_combine(x, sh, dinv):
    g = N_NODES // _BM
    hspec = pl.BlockSpec((_BM, DIM_H), lambda i: (i, 0))
    return pl.pallas_call(
        _combine_body,
        grid=(g,),
        in_specs=[pl.BlockSpec((_BM, DIM), lambda i: (i, 0))]
        + [hspec] * 2
        + [pl.BlockSpec((_BM, 1), lambda i: (i, 0))],
        out_specs=pl.BlockSpec((_BM, DIM), lambda i: (i, 0)),
        out_shape=jax.ShapeDtypeStruct((N_NODES, DIM), jnp.float32),
    )(x, *sh, dinv)


def kernel(edge_index, features, preference, W1, b1, W2, b2):
    row = edge_index[0].astype(jnp.int32)
    col = edge_index[1].astype(jnp.int32)
    padi = jnp.zeros((E_PAD - N_EDGES,), jnp.int32)
    row = jnp.concatenate([row, padi])
    col = jnp.concatenate([col, padi])

    rowp, degp0, degp1 = _sc_prep(row.reshape(E_ROWS, CH),
                                  col.reshape(E_ROWS, CH))
    ei = jnp.concatenate([rowp.reshape(-1), col]).reshape(2 * E_ROWS, CH)

    x_items = _tc_items(features, W1, b1, W2, b2)
    x_pref = _tc_pref(preference)
    x = jnp.concatenate([x_pref, x_items], axis=0)
    zpad = jnp.zeros((N_PAD - N_NODES, DIM), jnp.float32)
    xp = jnp.concatenate([x, zpad], axis=0)

    dinv_col, d2, y0, y1 = _tc_degscale(degp0, degp1, xp)

    _, _, st0, st1 = _sc_layers(y0, y1, ei, d2.reshape(N_PAD))

    x_hat = _tc_combine(x, (st0[:N_NODES], st1[:N_NODES]), dinv_col[:N_NODES])
    return (x_hat, preference)
